# Initial kernel scaffold; baseline (speedup 1.0000x reference)
import jax
import jax.numpy as jnp
from jax.experimental import pallas as pl


def kernel(depth_map_1, depth_map_2, translation_vectors, rotation_matrices, intrinsic_matrix):
    raise NotImplementedError("write your pallas kernel here")


# speedup vs baseline: 1.3511x; 1.3511x over previous
import functools
import jax
import jax.numpy as jnp
from jax import lax
from jax.experimental import pallas as pl
from jax.experimental.pallas import tpu as pltpu
from jax.experimental.pallas import tpu_sc as plsc

_B, _H, _W = 16, 512, 512
_HW = _H * _W
_N = _B * _HW
_NW = 32
_PPW = _N // _NW
_BLK = 1024
_NBLK = _PPW // _BLK
_VPB = _BLK // 16
_ROWS = _BLK // 128


def _make_warp_kernel():
    mesh = plsc.VectorSubcoreMesh(core_axis_name="c", subcore_axis_name="s")

    @functools.partial(
        pl.kernel,
        mesh=mesh,
        out_type=jax.ShapeDtypeStruct((_N,), jnp.float32),
        scratch_types=[
            pltpu.VMEM((16, 16), jnp.float32),
            pltpu.VMEM((_BLK,), jnp.float32),
            pltpu.VMEM((4, _ROWS, 128), jnp.int32),
            pltpu.VMEM((4, _ROWS, 128), jnp.float32),
            pltpu.VMEM((4, _BLK), jnp.float32),
            pltpu.VMEM((_BLK,), jnp.float32),
            pltpu.VMEM((_BLK,), jnp.float32),
            pltpu.SemaphoreType.DMA,
        ],
    )
    def warp(z1_hbm, z2_hbm, coef_hbm, out_hbm, cv, z1v, idxv, gv, hv, sv, ov, sem):
        wid = lax.axis_index("c") * 16 + lax.axis_index("s")
        b = wid // 2
        base = wid * _PPW
        boff = b * _HW
        pltpu.sync_copy(coef_hbm.at[b], cv)
        M00 = cv[0]; M01 = cv[1]; M02 = cv[2]
        M10 = cv[3]; M11 = cv[4]; M12 = cv[5]
        M20 = cv[6]; M21 = cv[7]; M22 = cv[8]
        Wv0 = cv[9]; Wv1 = cv[10]; Wv2 = cv[11]
        N0 = cv[12]; N1 = cv[13]; N2 = cv[14]
        W22 = cv[15]
        boffv = jnp.full((16,), boff, jnp.int32)
        lane = lax.iota(jnp.int32, 16)
        one = jnp.full((16,), 1.0, jnp.float32)

        def do_block(blk, carry):
            gb = base + blk * _BLK
            pltpu.sync_copy(z1_hbm.at[pl.ds(gb, _BLK)], z1v)
            lb = gb - boff

            def pass1(j, c2):
                off = lb + j * 16
                col = lax.rem(off, _W)
                row = lax.div(off, _W)
                u = (jnp.full((16,), col, jnp.int32) + lane).astype(jnp.float32)
                v = jnp.full((16,), row, jnp.int32).astype(jnp.float32)
                z1 = z1v[pl.ds(j * 16, 16)]
                p = M00 * u + M01 * v + M02
                q = M10 * u + M11 * v + M12
                r = M20 * u + M21 * v + M22
                zc = Wv2 + z1 * r
                inv = one / zc
                u2 = (z1 * p + Wv0) * inv
                v2 = (z1 * q + Wv1) * inv
                tx = u2.astype(jnp.int32)
                fx = jnp.where(tx.astype(jnp.float32) > u2, tx - 1, tx)
                ty = v2.astype(jnp.int32)
                fy = jnp.where(ty.astype(jnp.float32) > v2, ty - 1, ty)
                x0 = jnp.clip(fx, 0, _W - 1)
                x1 = jnp.clip(fx + 1, 0, _W - 1)
                y0 = jnp.clip(fy, 0, _H - 1)
                y1 = jnp.clip(fy + 1, 0, _H - 1)
                x0f = x0.astype(jnp.float32)
                x1f = x1.astype(jnp.float32)
                y0f = y0.astype(jnp.float32)
                y1f = y1.astype(jnp.float32)
                wx0 = x1f - u2
                wx1 = u2 - x0f
                wy0 = y1f - v2
                wy1 = v2 - y0f
                wa = wx0 * wy0
                wb = wx0 * wy1
                wc = wx1 * wy0
                wd = wx1 * wy1
                gx0 = N0 * x0f
                gx1 = N0 * x1f
                gy0 = N1 * y0f + N2
                gy1 = N1 * y1f + N2
                d = pl.ds(j * 16, 16)
                hv[0, d] = wa * (gx0 + gy0)
                hv[1, d] = wb * (gx0 + gy1)
                hv[2, d] = wc * (gx1 + gy0)
                hv[3, d] = wd * (gx1 + gy1)
                sv[d] = W22 * (wa + wb + wc + wd)
                yb0 = y0 * _W + boffv
                yb1 = y1 * _W + boffv
                r8 = j // 8
                c16 = lax.rem(j, 8) * 16
                dc = pl.ds(c16, 16)
                idxv[0, r8, dc] = yb0 + x0
                idxv[1, r8, dc] = yb1 + x0
                idxv[2, r8, dc] = yb0 + x1
                idxv[3, r8, dc] = yb1 + x1
                return c2

            lax.fori_loop(0, _VPB, pass1, 0)

            copies = []
            for c in range(4):
                for r in range(_ROWS):
                    copies.append(
                        pltpu.async_copy(z2_hbm.at[idxv.at[c, r]], gv.at[c, r], sem)
                    )
            for cp in copies:
                cp.wait()

            def pass2(j, c2):
                d = pl.ds(j * 16, 16)
                r8 = j // 8
                dc = pl.ds(lax.rem(j, 8) * 16, 16)
                acc = sv[d]
                acc = acc + hv[0, d] * gv[0, r8, dc]
                acc = acc + hv[1, d] * gv[1, r8, dc]
                acc = acc + hv[2, d] * gv[2, r8, dc]
                acc = acc + hv[3, d] * gv[3, r8, dc]
                ov[d] = acc
                return c2

            lax.fori_loop(0, _VPB, pass2, 0)
            pltpu.sync_copy(ov, out_hbm.at[pl.ds(gb, _BLK)])
            return carry

        lax.fori_loop(0, _NBLK, do_block, 0)

    return warp


_warp = _make_warp_kernel()


@jax.jit
def kernel(depth_map_1, depth_map_2, translation_vectors, rotation_matrices, intrinsic_matrix):
    K = intrinsic_matrix
    Ki = jnp.linalg.inv(K)
    Rt = jnp.swapaxes(rotation_matrices, 1, 2)
    temp = jnp.einsum('ij,bjk->bik', K, Rt)
    Wv = jnp.einsum('bij,bjk->bik', temp, -translation_vectors)[..., 0]
    M = jnp.einsum('bij,jk->bik', temp, Ki)
    W2 = jnp.einsum('ij,bjk->bik', K, translation_vectors)[:, 2, 0]
    temp2 = jnp.einsum('ij,bjk->bik', K, rotation_matrices)
    M2 = jnp.einsum('bij,jk->bik', temp2, Ki)
    Nr = M2[:, 2, :]
    scal = jnp.stack(
        [M[:, 0, 0], M[:, 0, 1], M[:, 0, 2],
         M[:, 1, 0], M[:, 1, 1], M[:, 1, 2],
         M[:, 2, 0], M[:, 2, 1], M[:, 2, 2],
         Wv[:, 0], Wv[:, 1], Wv[:, 2],
         Nr[:, 0], Nr[:, 1], Nr[:, 2],
         W2], axis=1).astype(jnp.float32)
    coef = jnp.tile(scal[:, :, None], (1, 1, 16))
    z1f = depth_map_1.reshape(_N)
    z2f = depth_map_2.reshape(_N)
    out = _warp(z1f, z2f, coef)
    return out.reshape(_B, _H, _W, 1)


# speedup vs baseline: 1.6980x; 1.2567x over previous
import functools
import jax
import jax.numpy as jnp
from jax import lax
from jax.experimental import pallas as pl
from jax.experimental.pallas import tpu as pltpu
from jax.experimental.pallas import tpu_sc as plsc

_B, _H, _W = 16, 512, 512
_HW = _H * _W
_N = _B * _HW
_NW = 32
_PPW = _N // _NW
_BLK = 1024
_NBLK = _PPW // _BLK
_VPB = _BLK // 16
_NROW = 4 * (_BLK // 128)


def _make_warp_kernel():
    mesh = plsc.VectorSubcoreMesh(core_axis_name="c", subcore_axis_name="s")

    @functools.partial(
        pl.kernel,
        mesh=mesh,
        out_type=jax.ShapeDtypeStruct((_N,), jnp.float32),
        scratch_types=[
            pltpu.VMEM((16, 16), jnp.float32),
            pltpu.VMEM((2, _BLK), jnp.float32),
            pltpu.VMEM((2, _NROW, 128), jnp.int32),
            pltpu.VMEM((2, _NROW, 128), jnp.float32),
            pltpu.VMEM((2, 4, _BLK), jnp.float32),
            pltpu.VMEM((2, _BLK), jnp.float32),
            pltpu.VMEM((_BLK,), jnp.float32),
            pltpu.SemaphoreType.DMA,
        ],
    )
    def warp(z1_hbm, z2_hbm, coef_hbm, out_hbm, cv, z1v, idxv, gv, hv, sv, ov, sem):
        wid = lax.axis_index("c") * 16 + lax.axis_index("s")
        b = wid // 2
        base = wid * _PPW
        boff = b * _HW
        pltpu.sync_copy(coef_hbm.at[b], cv)
        M00 = cv[0]; M01 = cv[1]; M02 = cv[2]
        M10 = cv[3]; M11 = cv[4]; M12 = cv[5]
        M20 = cv[6]; M21 = cv[7]; M22 = cv[8]
        Wv0 = cv[9]; Wv1 = cv[10]; Wv2 = cv[11]
        N0 = cv[12]; N1 = cv[13]; N2 = cv[14]
        W22 = cv[15]
        boffv = jnp.full((16,), boff, jnp.int32)
        lane = lax.iota(jnp.int32, 16)
        one = jnp.full((16,), 1.0, jnp.float32)

        def load_z1(k, p):
            pltpu.sync_copy(z1_hbm.at[pl.ds(base + k * _BLK, _BLK)], z1v.at[p])

        def pass1(k, p):
            lb = base + k * _BLK - boff

            def body(j, c2):
                off = lb + j * 16
                col = lax.rem(off, _W)
                row = lax.div(off, _W)
                u = (jnp.full((16,), col, jnp.int32) + lane).astype(jnp.float32)
                v = jnp.full((16,), row, jnp.int32).astype(jnp.float32)
                z1 = z1v[p, pl.ds(j * 16, 16)]
                pp = M00 * u + M01 * v + M02
                qq = M10 * u + M11 * v + M12
                rr = M20 * u + M21 * v + M22
                zc = Wv2 + z1 * rr
                inv = one / zc
                u2 = (z1 * pp + Wv0) * inv
                v2 = (z1 * qq + Wv1) * inv
                tx = u2.astype(jnp.int32)
                fx = jnp.where(tx.astype(jnp.float32) > u2, tx - 1, tx)
                ty = v2.astype(jnp.int32)
                fy = jnp.where(ty.astype(jnp.float32) > v2, ty - 1, ty)
                x0 = jnp.clip(fx, 0, _W - 1)
                x1 = jnp.clip(fx + 1, 0, _W - 1)
                y0 = jnp.clip(fy, 0, _H - 1)
                y1 = jnp.clip(fy + 1, 0, _H - 1)
                x0f = x0.astype(jnp.float32)
                x1f = x1.astype(jnp.float32)
                y0f = y0.astype(jnp.float32)
                y1f = y1.astype(jnp.float32)
                wx0 = x1f - u2
                wx1 = u2 - x0f
                wy0 = y1f - v2
                wy1 = v2 - y0f
                wa = wx0 * wy0
                wb = wx0 * wy1
                wc = wx1 * wy0
                wd = wx1 * wy1
                gx0 = N0 * x0f
                gx1 = N0 * x1f
                gy0 = N1 * y0f + N2
                gy1 = N1 * y1f + N2
                d = pl.ds(j * 16, 16)
                hv[p, 0, d] = wa * (gx0 + gy0)
                hv[p, 1, d] = wb * (gx0 + gy1)
                hv[p, 2, d] = wc * (gx1 + gy0)
                hv[p, 3, d] = wd * (gx1 + gy1)
                sv[p, d] = W22 * (wa + wb + wc + wd)
                yb0 = y0 * _W + boffv
                yb1 = y1 * _W + boffv
                r8 = j // 8
                dc = pl.ds(lax.rem(j, 8) * 16, 16)
                idxv[p, r8, dc] = yb0 + x0
                idxv[p, 8 + r8, dc] = yb1 + x0
                idxv[p, 16 + r8, dc] = yb0 + x1
                idxv[p, 24 + r8, dc] = yb1 + x1
                return c2

            lax.fori_loop(0, _VPB, body, 0)

        def fire(p):
            for r in range(_NROW):
                pltpu.async_copy(z2_hbm.at[idxv.at[p, r]], gv.at[p, r], sem)

        def drain(p):
            for r in range(_NROW):
                pltpu.make_async_copy(z2_hbm.at[pl.ds(0, 128)], gv.at[p, r], sem).wait()

        def pass2(k, p):
            def body(j, c2):
                d = pl.ds(j * 16, 16)
                r8 = j // 8
                dc = pl.ds(lax.rem(j, 8) * 16, 16)
                acc = sv[p, d]
                acc = acc + hv[p, 0, d] * gv[p, r8, dc]
                acc = acc + hv[p, 1, d] * gv[p, 8 + r8, dc]
                acc = acc + hv[p, 2, d] * gv[p, 16 + r8, dc]
                acc = acc + hv[p, 3, d] * gv[p, 24 + r8, dc]
                ov[d] = acc
                return c2

            lax.fori_loop(0, _VPB, body, 0)
            pltpu.sync_copy(ov, out_hbm.at[pl.ds(base + k * _BLK, _BLK)])

        load_z1(0, 0)
        pass1(0, 0)
        fire(0)

        def body(k, carry):
            p = lax.rem(k, 2)
            q = 1 - p
            load_z1(k + 1, q)
            pass1(k + 1, q)
            drain(p)
            fire(q)
            pass2(k, p)
            return carry

        lax.fori_loop(0, _NBLK - 1, body, 0)
        pL = (_NBLK - 1) % 2
        drain(pL)
        pass2(_NBLK - 1, pL)

    return warp


_warp = _make_warp_kernel()


@jax.jit
def kernel(depth_map_1, depth_map_2, translation_vectors, rotation_matrices, intrinsic_matrix):
    K = intrinsic_matrix
    Ki = jnp.linalg.inv(K)
    Rt = jnp.swapaxes(rotation_matrices, 1, 2)
    temp = jnp.einsum('ij,bjk->bik', K, Rt)
    Wv = jnp.einsum('bij,bjk->bik', temp, -translation_vectors)[..., 0]
    M = jnp.einsum('bij,jk->bik', temp, Ki)
    W2 = jnp.einsum('ij,bjk->bik', K, translation_vectors)[:, 2, 0]
    temp2 = jnp.einsum('ij,bjk->bik', K, rotation_matrices)
    M2 = jnp.einsum('bij,jk->bik', temp2, Ki)
    Nr = M2[:, 2, :]
    scal = jnp.stack(
        [M[:, 0, 0], M[:, 0, 1], M[:, 0, 2],
         M[:, 1, 0], M[:, 1, 1], M[:, 1, 2],
         M[:, 2, 0], M[:, 2, 1], M[:, 2, 2],
         Wv[:, 0], Wv[:, 1], Wv[:, 2],
         Nr[:, 0], Nr[:, 1], Nr[:, 2],
         W2], axis=1).astype(jnp.float32)
    coef = jnp.tile(scal[:, :, None], (1, 1, 16))
    z1f = depth_map_1.reshape(_N)
    z2f = depth_map_2.reshape(_N)
    out = _warp(z1f, z2f, coef)
    return out.reshape(_B, _H, _W, 1)


# speedup vs baseline: 1.7122x; 1.0083x over previous
import functools
import jax
import jax.numpy as jnp
from jax import lax
from jax.experimental import pallas as pl
from jax.experimental.pallas import tpu as pltpu
from jax.experimental.pallas import tpu_sc as plsc

_B, _H, _W = 16, 512, 512
_HW = _H * _W
_N = _B * _HW
_NW = 32
_PPW = _N // _NW
_BLK = 1024
_NBLK = _PPW // _BLK
_VPB = _BLK // 16
_NROW = 4 * (_BLK // 128)


def _make_warp_kernel():
    mesh = plsc.VectorSubcoreMesh(core_axis_name="c", subcore_axis_name="s")

    @functools.partial(
        pl.kernel,
        mesh=mesh,
        out_type=jax.ShapeDtypeStruct((_N,), jnp.float32),
        scratch_types=[
            pltpu.VMEM((16, 16), jnp.float32),
            pltpu.VMEM((2, _BLK), jnp.float32),
            pltpu.VMEM((2, _NROW, 128), jnp.int32),
            pltpu.VMEM((2, _NROW, 128), jnp.float32),
            pltpu.VMEM((2, 4, _BLK), jnp.float32),
            pltpu.VMEM((2, _BLK), jnp.float32),
            pltpu.VMEM((_BLK,), jnp.float32),
            pltpu.SemaphoreType.DMA,
        ],
    )
    def warp(z1_hbm, z2_hbm, coef_hbm, out_hbm, cv, z1v, idxv, gv, hv, sv, ov, sem):
        wid = lax.axis_index("c") * 16 + lax.axis_index("s")
        b = wid // 2
        base = wid * _PPW
        boff = b * _HW
        pltpu.sync_copy(coef_hbm.at[b], cv)
        M00 = cv[0]; M01 = cv[1]; M02 = cv[2]
        M10 = cv[3]; M11 = cv[4]; M12 = cv[5]
        M20 = cv[6]; M21 = cv[7]; M22 = cv[8]
        Wv0 = cv[9]; Wv1 = cv[10]; Wv2 = cv[11]
        N0 = cv[12]; N1 = cv[13]; N2 = cv[14]
        W22 = cv[15]
        boffv = jnp.full((16,), boff, jnp.int32)
        lane = lax.iota(jnp.int32, 16)
        one = jnp.full((16,), 1.0, jnp.float32)

        def load_z1(k, p):
            pltpu.sync_copy(z1_hbm.at[pl.ds(base + k * _BLK, _BLK)], z1v.at[p])

        def pass1(k, p):
            lb = base + k * _BLK - boff

            def emit1(j):
                off = lb + (j << 4)
                col = off & (_W - 1)
                row = off >> 9
                d = pl.ds(pl.multiple_of(j << 4, 16), 16)
                dc = pl.ds(pl.multiple_of((j & 7) << 4, 16), 16)
                u = (jnp.full((16,), col, jnp.int32) + lane).astype(jnp.float32)
                v = jnp.full((16,), row, jnp.int32).astype(jnp.float32)
                z1 = z1v[p, d]
                pp = M00 * u + M01 * v + M02
                qq = M10 * u + M11 * v + M12
                rr = M20 * u + M21 * v + M22
                zc = Wv2 + z1 * rr
                inv = one / zc
                u2 = (z1 * pp + Wv0) * inv
                v2 = (z1 * qq + Wv1) * inv
                tx = u2.astype(jnp.int32)
                fx = jnp.where(tx.astype(jnp.float32) > u2, tx - 1, tx)
                ty = v2.astype(jnp.int32)
                fy = jnp.where(ty.astype(jnp.float32) > v2, ty - 1, ty)
                x0 = jnp.clip(fx, 0, _W - 1)
                x1 = jnp.clip(fx + 1, 0, _W - 1)
                y0 = jnp.clip(fy, 0, _H - 1)
                y1 = jnp.clip(fy + 1, 0, _H - 1)
                x0f = x0.astype(jnp.float32)
                x1f = x1.astype(jnp.float32)
                y0f = y0.astype(jnp.float32)
                y1f = y1.astype(jnp.float32)
                wx0 = x1f - u2
                wx1 = u2 - x0f
                wy0 = y1f - v2
                wy1 = v2 - y0f
                wa = wx0 * wy0
                wb = wx0 * wy1
                wc = wx1 * wy0
                wd = wx1 * wy1
                gx0 = N0 * x0f
                gx1 = N0 * x1f
                gy0 = N1 * y0f + N2
                gy1 = N1 * y1f + N2
                hv[p, 0, d] = wa * (gx0 + gy0)
                hv[p, 1, d] = wb * (gx0 + gy1)
                hv[p, 2, d] = wc * (gx1 + gy0)
                hv[p, 3, d] = wd * (gx1 + gy1)
                sv[p, d] = W22 * (wa + wb + wc + wd)
                yb0 = y0 * _W + boffv
                yb1 = y1 * _W + boffv
                r8 = j >> 3
                idxv[p, r8, dc] = yb0 + x0
                idxv[p, 8 + r8, dc] = yb1 + x0
                idxv[p, 16 + r8, dc] = yb0 + x1
                idxv[p, 24 + r8, dc] = yb1 + x1

            def body(j2, c2):
                emit1(j2 * 2)
                emit1(j2 * 2 + 1)
                return c2

            lax.fori_loop(0, _VPB // 2, body, 0)

        def fire(p):
            for r in range(_NROW):
                pltpu.async_copy(z2_hbm.at[idxv.at[p, r]], gv.at[p, r], sem)

        def drain(p):
            for r in range(_NROW):
                pltpu.make_async_copy(z2_hbm.at[pl.ds(0, 128)], gv.at[p, r], sem).wait()

        def pass2(k, p):
            def emit2(j):
                d = pl.ds(pl.multiple_of(j << 4, 16), 16)
                r8 = j >> 3
                dc = pl.ds(pl.multiple_of((j & 7) << 4, 16), 16)
                acc = sv[p, d]
                acc = acc + hv[p, 0, d] * gv[p, r8, dc]
                acc = acc + hv[p, 1, d] * gv[p, 8 + r8, dc]
                acc = acc + hv[p, 2, d] * gv[p, 16 + r8, dc]
                acc = acc + hv[p, 3, d] * gv[p, 24 + r8, dc]
                ov[d] = acc

            def body(j2, c2):
                emit2(j2 * 2)
                emit2(j2 * 2 + 1)
                return c2

            lax.fori_loop(0, _VPB // 2, body, 0)
            pltpu.sync_copy(ov, out_hbm.at[pl.ds(base + k * _BLK, _BLK)])

        load_z1(0, 0)
        pass1(0, 0)
        fire(0)

        def body(k, carry):
            p = lax.rem(k, 2)
            q = 1 - p
            load_z1(k + 1, q)
            pass1(k + 1, q)
            drain(p)
            fire(q)
            pass2(k, p)
            return carry

        lax.fori_loop(0, _NBLK - 1, body, 0)
        pL = (_NBLK - 1) % 2
        drain(pL)
        pass2(_NBLK - 1, pL)

    return warp


_warp = _make_warp_kernel()


@jax.jit
def kernel(depth_map_1, depth_map_2, translation_vectors, rotation_matrices, intrinsic_matrix):
    K = intrinsic_matrix
    Ki = jnp.linalg.inv(K)
    Rt = jnp.swapaxes(rotation_matrices, 1, 2)
    temp = jnp.einsum('ij,bjk->bik', K, Rt)
    Wv = jnp.einsum('bij,bjk->bik', temp, -translation_vectors)[..., 0]
    M = jnp.einsum('bij,jk->bik', temp, Ki)
    W2 = jnp.einsum('ij,bjk->bik', K, translation_vectors)[:, 2, 0]
    temp2 = jnp.einsum('ij,bjk->bik', K, rotation_matrices)
    M2 = jnp.einsum('bij,jk->bik', temp2, Ki)
    Nr = M2[:, 2, :]
    scal = jnp.stack(
        [M[:, 0, 0], M[:, 0, 1], M[:, 0, 2],
         M[:, 1, 0], M[:, 1, 1], M[:, 1, 2],
         M[:, 2, 0], M[:, 2, 1], M[:, 2, 2],
         Wv[:, 0], Wv[:, 1], Wv[:, 2],
         Nr[:, 0], Nr[:, 1], Nr[:, 2],
         W2], axis=1).astype(jnp.float32)
    coef = jnp.tile(scal[:, :, None], (1, 1, 16))
    z1f = depth_map_1.reshape(_N)
    z2f = depth_map_2.reshape(_N)
    out = _warp(z1f, z2f, coef)
    return out.reshape(_B, _H, _W, 1)
